# baseline (speedup 1.0000x reference)
import jax
import jax.numpy as jnp
from jax.experimental import pallas as pl

_NDIM = 3
_EPS = 1e-5


def _leaky(x):
    return jnp.where(x >= 0, x, 0.01 * x)


def _bn(x, g, b):
    mu = jnp.mean(x, axis=0, keepdims=True)
    var = jnp.mean((x - mu) ** 2, axis=0, keepdims=True)
    return g * ((x - mu) * jax.lax.rsqrt(var + _EPS)) + b


def _dense(x, w, b):
    return jnp.dot(x, w, preferred_element_type=jnp.float32) + b


def _forward_kernel(pos_ref, w_ref, *refs):
    args = [r[...] for r in refs[:-2]]
    out_pos, out_w = refs[-2], refs[-1]

    it = iter(args)

    def take(n):
        return [next(it) for _ in range(n)]

    riW0, riB0, riG0, riBt0, riW1, riB1, riG1, riBt1 = take(8)

    x = w_ref[...]
    x = _leaky(_bn(_dense(x, riW0, riB0), riG0, riBt0))
    w = _leaky(_bn(_dense(x, riW1, riB1), riG1, riBt1))
    pos = pos_ref[...]

    for _ in range(2):
        dW0, dB0, dG0, dBt0, dW1p, dB1p, dW1w, dB1w = take(8)
        h = _leaky(_bn(_dense(w, dW0, dB0), dG0, dBt0))
        pos = pos + _dense(h, dW1p, dB1p)
        w = w + _dense(h, dW1w, dB1w)

    roW0, roB0, roG0, roBt0, roW1, roB1 = take(6)
    h = _leaky(_bn(_dense(w, roW0, roB0), roG0, roBt0))
    out_pos[...] = pos
    out_w[...] = _dense(h, roW1, roB1)


def _row(v):
    return v.reshape(1, -1)


def kernel(positions, weights, params, batch):
    del batch
    n = positions.shape[0]

    flat = []
    for p in params["readin"]:
        flat += [p["W"], _row(p["b"]), _row(p["gamma"]), _row(p["beta"])]
    for blk in params["blocks"]:
        l0, l1 = blk["delta"]
        flat += [l0["W"], _row(l0["b"]), _row(l0["gamma"]), _row(l0["beta"])]
        flat += [l1["W"][:, :_NDIM], _row(l1["b"][:_NDIM]),
                 l1["W"][:, _NDIM:], _row(l1["b"][_NDIM:])]
    ro0, ro1 = params["readout"]
    flat += [ro0["W"], _row(ro0["b"]), _row(ro0["gamma"]), _row(ro0["beta"])]
    flat += [ro1["W"], _row(ro1["b"])]

    out_dim = ro1["W"].shape[1]
    pos_out, w_out = pl.pallas_call(
        _forward_kernel,
        out_shape=(
            jax.ShapeDtypeStruct((n, _NDIM), jnp.float32),
            jax.ShapeDtypeStruct((n, out_dim), jnp.float32),
        ),
    )(positions, weights, *flat)
    return pos_out, w_out


# speedup vs baseline: 1.1878x; 1.1878x over previous
import jax
import jax.numpy as jnp
from jax.experimental import pallas as pl

_NDIM = 3
_EPS = 1e-5


def _leaky(x):
    return jnp.where(x >= 0, x, 0.01 * x)


def _bn(x, g, b):
    mu = jnp.mean(x, axis=0, keepdims=True)
    var = jnp.mean((x - mu) ** 2, axis=0, keepdims=True)
    return g * ((x - mu) * jax.lax.rsqrt(var + _EPS)) + b


def _dense(x, w, b):
    return jnp.dot(x, w, preferred_element_type=jnp.float32) + b


def _forward_kernel(pos_ref, w_ref, *refs):
    args = [r[...] for r in refs[:-2]]
    out_pos, out_w = refs[-2], refs[-1]

    it = iter(args)

    def take(n):
        return [next(it) for _ in range(n)]

    riW0, riB0, riG0, riBt0, riW1, riB1, riG1, riBt1 = take(8)

    x = w_ref[...]
    x = _leaky(_bn(_dense(x, riW0, riB0), riG0, riBt0))
    w = _leaky(_bn(_dense(x, riW1, riB1), riG1, riBt1))
    pos = pos_ref[...]

    for _ in range(2):
        dW0, dB0, dG0, dBt0, dW1, dB1 = take(6)
        h = _leaky(_bn(_dense(w, dW0, dB0), dG0, dBt0))
        pos = pos + _dense(h, dW1[:, :_NDIM], dB1[:_NDIM])
        w = w + _dense(h, dW1[:, _NDIM:], dB1[_NDIM:])

    roW0, roB0, roG0, roBt0, roW1, roB1 = take(6)
    h = _leaky(_bn(_dense(w, roW0, roB0), roG0, roBt0))
    out_pos[...] = pos
    out_w[...] = _dense(h, roW1, roB1)


def kernel(positions, weights, params, batch):
    del batch
    n = positions.shape[0]

    flat = []
    for p in params["readin"]:
        flat += [p["W"], p["b"], p["gamma"], p["beta"]]
    for blk in params["blocks"]:
        l0, l1 = blk["delta"]
        flat += [l0["W"], l0["b"], l0["gamma"], l0["beta"], l1["W"], l1["b"]]
    ro0, ro1 = params["readout"]
    flat += [ro0["W"], ro0["b"], ro0["gamma"], ro0["beta"], ro1["W"], ro1["b"]]

    out_dim = ro1["W"].shape[1]
    pos_out, w_out = pl.pallas_call(
        _forward_kernel,
        out_shape=(
            jax.ShapeDtypeStruct((n, _NDIM), jnp.float32),
            jax.ShapeDtypeStruct((n, out_dim), jnp.float32),
        ),
    )(positions, weights, *flat)
    return pos_out, w_out
